# baseline (speedup 1.0000x reference)
import functools

import jax
import jax.numpy as jnp
from jax import lax
from jax.experimental import pallas as pl
from jax.experimental.pallas import tpu as pltpu
from jax.experimental.pallas import tpu_sc as plsc

N = 10000
D = 128
E = 320000

NC = 2
NS = 16
NW = NC * NS
EPW = E // NW
CHUNK = 80
NCHUNK = EPW // CHUNK
NP = 10240
NPS = NP // NS
RING = 3
PEEL = NCHUNK % RING

_mesh = plsc.VectorSubcoreMesh(core_axis_name="c", subcore_axis_name="s")
_sc_params = pltpu.CompilerParams(use_tc_tiling_on_sc=False,
                                 needs_layout_passes=False)


def _dma_sems():
    return [pltpu.SemaphoreType.DMA for _ in range(RING)]


@functools.partial(
    pl.kernel,
    out_type=jax.ShapeDtypeStruct((NC * NP, 16), jnp.float32),
    mesh=_mesh,
    compiler_params=_sc_params,
    scratch_types=[
        pltpu.VMEM_SHARED((NP, 16), jnp.float32),
        [pltpu.VMEM((CHUNK,), jnp.int32) for _ in range(RING)],
        pltpu.VMEM((CHUNK, 16), jnp.float32),
        pltpu.VMEM((NPS, 16), jnp.float32),
        _dma_sems(),
    ],
)
def _deg_kernel(col_hbm, zeros_hbm, ones_hbm, out_hbm,
                degp, cidx, ones, zbuf, isem):
    cid = lax.axis_index("c")
    sid = lax.axis_index("s")
    wid = cid * NS + sid
    e0 = wid * EPW
    last = NCHUNK - 1
    pltpu.sync_copy(zeros_hbm, zbuf)
    pltpu.sync_copy(ones_hbm, ones)
    pltpu.sync_copy(zbuf, degp.at[pl.ds(sid * NPS, NPS)])
    plsc.subcore_barrier()

    def idx_start(i, s):
        pltpu.async_copy(col_hbm.at[pl.ds(e0 + i * CHUNK, CHUNK)],
                         cidx[s], isem[s])

    def idx_wait(s):
        pltpu.make_async_copy(col_hbm.at[pl.ds(e0, CHUNK)],
                              cidx[s], isem[s]).wait()

    for s in range(RING):
        idx_start(s, s)

    def step(i, s):
        idx_wait(s)
        pltpu.sync_copy(ones, degp.at[cidx[s]], add=True)
        idx_start(lax.min(i + RING, last), s)

    def body(io, carry):
        for b in range(RING):
            step(io * RING + b, b)
        return carry

    lax.fori_loop(0, (NCHUNK - PEEL) // RING, body, 0)
    for p in range(PEEL):
        i = NCHUNK - PEEL + p
        s = i % RING
        idx_wait(s)
        pltpu.sync_copy(ones, degp.at[cidx[s]], add=True)
    for p in range(PEEL, RING):
        idx_wait((NCHUNK - PEEL + p) % RING)
    plsc.subcore_barrier()
    pltpu.sync_copy(degp.at[pl.ds(sid * NPS, NPS)],
                    out_hbm.at[pl.ds(cid * NP + sid * NPS, NPS)])


@functools.partial(
    pl.kernel,
    out_type=jax.ShapeDtypeStruct((NC * NP, D), jnp.float32),
    mesh=_mesh,
    compiler_params=_sc_params,
    scratch_types=[
        pltpu.VMEM_SHARED((NP, D), jnp.float32),
        [pltpu.VMEM((CHUNK,), jnp.int32) for _ in range(RING)],
        [pltpu.VMEM((CHUNK,), jnp.int32) for _ in range(RING)],
        [pltpu.VMEM((CHUNK, D), jnp.float32) for _ in range(RING)],
        pltpu.VMEM((CHUNK, D), jnp.float32),
        _dma_sems(),
        _dma_sems(),
        _dma_sems(),
        _dma_sems(),
    ],
)
def _agg_kernel(y_hbm, row_hbm, col_hbm, zeros_hbm, out_hbm,
                acc, ridx, cidx, rows, zbuf, rsem, csem, gsem, ssem):
    cid = lax.axis_index("c")
    sid = lax.axis_index("s")
    wid = cid * NS + sid
    e0 = wid * EPW
    last = NCHUNK - 1
    pltpu.sync_copy(zeros_hbm, zbuf)
    for j in range(NPS // CHUNK):
        pltpu.sync_copy(zbuf, acc.at[pl.ds(sid * NPS + j * CHUNK, CHUNK)])
    plsc.subcore_barrier()

    def ridx_start(i, s):
        pltpu.async_copy(row_hbm.at[pl.ds(e0 + i * CHUNK, CHUNK)],
                         ridx[s], rsem[s])

    def ridx_wait(s):
        pltpu.make_async_copy(row_hbm.at[pl.ds(e0, CHUNK)],
                              ridx[s], rsem[s]).wait()

    def cidx_start(i, s):
        pltpu.async_copy(col_hbm.at[pl.ds(e0 + i * CHUNK, CHUNK)],
                         cidx[s], csem[s])

    def cidx_wait(s):
        pltpu.make_async_copy(col_hbm.at[pl.ds(e0, CHUNK)],
                              cidx[s], csem[s]).wait()

    def gather_start(s):
        pltpu.async_copy(y_hbm.at[ridx[s]], rows[s], gsem[s])

    def gather_wait(s):
        pltpu.make_async_copy(y_hbm.at[ridx[s]], rows[s], gsem[s]).wait()

    def scatter_start(s):
        pltpu.async_copy(rows[s], acc.at[cidx[s]], ssem[s], add=True)

    def scatter_wait(s):
        pltpu.make_async_copy(rows[s], acc.at[cidx[s]], ssem[s]).wait()

    for s in range(RING):
        ridx_start(s, s)
    cidx_start(0, 0)
    cidx_start(1, 1)
    ridx_wait(0)
    gather_start(0)
    ridx_wait(1)
    gather_start(1)

    gather_wait(0)
    ridx_wait(2)
    gather_start(2)
    cidx_wait(0)
    scatter_start(0)
    ridx_start(3, 0)
    cidx_start(2, 2)

    gather_wait(1)
    ridx_wait(0)
    scatter_wait(0)
    gather_start(0)
    cidx_wait(1)
    scatter_start(1)
    ridx_start(4, 1)
    cidx_start(3, 0)

    def step(i, s):
        s1 = (s + 2) % RING
        gather_wait(s)
        ridx_wait(s1)
        scatter_wait(s1)
        gather_start(s1)
        cidx_wait(s)
        scatter_start(s)
        ridx_start(lax.min(i + 3, last), s)
        cidx_start(lax.min(i + 2, last), s1)

    def body(io, carry):
        for b in range(RING):
            i = 2 + io * RING + b
            step(i, (2 + b) % RING)
        return carry

    lax.fori_loop(0, (NCHUNK - 2) // RING, body, 0)
    scatter_wait((NCHUNK - 1) % RING)
    gather_wait((NCHUNK + 1) % RING)
    gather_wait(NCHUNK % RING)
    ridx_wait((NCHUNK + 2) % RING)
    cidx_wait(NCHUNK % RING)
    cidx_wait((NCHUNK + 1) % RING)
    plsc.subcore_barrier()
    pltpu.sync_copy(acc.at[pl.ds(sid * NPS, NPS)],
                    out_hbm.at[pl.ds(cid * NP + sid * NPS, NPS)])


_EU = 4
RING_E = 3
PEEL_E = NCHUNK % RING_E


@functools.partial(
    pl.kernel,
    out_type=jax.ShapeDtypeStruct((E,), jnp.float32),
    mesh=_mesh,
    compiler_params=_sc_params,
    scratch_types=[
        pltpu.VMEM_SHARED((NP, D), jnp.bfloat16),
        [pltpu.VMEM((CHUNK,), jnp.int32) for _ in range(RING_E)],
        [pltpu.VMEM((CHUNK,), jnp.int32) for _ in range(RING_E)],
        [pltpu.VMEM((CHUNK, D), jnp.bfloat16) for _ in range(RING_E)],
        [pltpu.VMEM((CHUNK, D), jnp.bfloat16) for _ in range(RING_E)],
        [pltpu.VMEM((CHUNK,), jnp.float32) for _ in range(RING_E)],
        [pltpu.SemaphoreType.DMA for _ in range(RING_E)],
        [pltpu.SemaphoreType.DMA for _ in range(RING_E)],
        [pltpu.SemaphoreType.DMA for _ in range(RING_E)],
    ],
)
def _edge_kernel(h_hbm, row_hbm, col_hbm, out_hbm,
                 hsh, ridx, cidx, rr, rc, sbuf, isem, gsem, osem):
    cid = lax.axis_index("c")
    sid = lax.axis_index("s")
    wid = cid * NS + sid
    e0 = wid * EPW
    last = NCHUNK - 1
    lane = lax.iota(jnp.int32, 16)
    pltpu.sync_copy(h_hbm.at[pl.ds(sid * NPS, NPS)],
                    hsh.at[pl.ds(sid * NPS, NPS)])
    plsc.subcore_barrier()

    def idx_start(i, s):
        pltpu.async_copy(row_hbm.at[pl.ds(e0 + i * CHUNK, CHUNK)],
                         ridx[s], isem[s])
        pltpu.async_copy(col_hbm.at[pl.ds(e0 + i * CHUNK, CHUNK)],
                         cidx[s], isem[s])

    def idx_wait(s):
        pltpu.make_async_copy(row_hbm.at[pl.ds(e0, CHUNK)],
                              ridx[s], isem[s]).wait()
        pltpu.make_async_copy(col_hbm.at[pl.ds(e0, CHUNK)],
                              cidx[s], isem[s]).wait()

    def gather_start(s):
        pltpu.async_copy(hsh.at[ridx[s]], rr[s], gsem[s])
        pltpu.async_copy(hsh.at[cidx[s]], rc[s], gsem[s])

    def gather_wait(s):
        pltpu.make_async_copy(hsh.at[ridx[s]], rr[s], gsem[s]).wait()
        pltpu.make_async_copy(hsh.at[cidx[s]], rc[s], gsem[s]).wait()

    dnums = lax.GatherDimensionNumbers(
        offset_dims=(), collapsed_slice_dims=(0,), start_index_map=(0,))

    def allsum(v):
        for sh in (8, 4, 2, 1):
            p = lax.gather(v, (lane ^ sh)[:, None], dnums, slice_sizes=(1,),
                           mode=lax.GatherScatterMode.PROMISE_IN_BOUNDS)
            v = v + p
        return v

    himask = jnp.full((16,), -65536, jnp.int32)

    def split_bf16(x32):
        xi = plsc.bitcast(x32, jnp.int32)
        lo = plsc.bitcast(lax.shift_left(xi, 16), jnp.float32)
        hi = plsc.bitcast(xi & himask, jnp.float32)
        return lo, hi

    def compute(s):
        def group_body(g, carry2):
            def edge_body(t, acc16):
                for u in range(_EU):
                    e = g * 16 + t * _EU + u
                    v = jnp.zeros((16,), jnp.float32)
                    for k in range(D // 32):
                        a0, a1 = split_bf16(rr[s][e, pl.ds(k * 32, 32)])
                        b0, b1 = split_bf16(rc[s][e, pl.ds(k * 32, 32)])
                        v = v + a0 * b0
                        v = v + a1 * b1
                    acc16 = jnp.where(lane == t * _EU + u, allsum(v), acc16)
                return acc16

            z = lax.fori_loop(0, 16 // _EU, edge_body,
                              jnp.zeros((16,), jnp.float32))
            sbuf[s][pl.ds(g * 16, 16)] = 1.0 / (1.0 + jnp.exp(-z))
            return carry2

        lax.fori_loop(0, CHUNK // 16, group_body, 0)

    def out_start(i, s):
        pltpu.async_copy(sbuf[s], out_hbm.at[pl.ds(e0 + i * CHUNK, CHUNK)],
                         osem[s])

    def out_wait(s):
        pltpu.make_async_copy(sbuf[s], out_hbm.at[pl.ds(e0, CHUNK)],
                              osem[s]).wait()

    for s in range(RING_E):
        idx_start(s, s)
    for s in range(RING_E - 1):
        idx_wait(s)
        gather_start(s)

    def step(i, s, first):
        s1 = (s + RING_E - 1) % RING_E
        gather_wait(s)
        idx_wait(s1)
        gather_start(s1)
        if not first:
            out_wait(s)
        compute(s)
        out_start(i, s)
        idx_start(lax.min(i + RING_E, last), s)

    for s in range(RING_E):
        step(s, s, True)

    def body(io, carry):
        for b in range(RING_E):
            step(RING_E + io * RING_E + b, b, False)
        return carry

    lax.fori_loop(0, (NCHUNK - RING_E - PEEL_E) // RING_E, body, 0)
    for p in range(PEEL_E):
        i = NCHUNK - PEEL_E + p
        s = i % RING_E
        gather_wait(s)
        out_wait(s)
        compute(s)
        out_start(i, s)
    for p in range(PEEL_E, RING_E):
        idx_wait((NCHUNK - PEEL_E + p) % RING_E)
    for s in range(RING_E):
        out_wait((NCHUNK - PEEL_E - 1 + ((s + 1))) % RING_E)


def _dis_from(degp_ref):
    deg = degp_ref[:N, 0:1] + degp_ref[NP:NP + N, 0:1]
    safe = jnp.where(deg > 0, deg, 1.0)
    return jnp.where(deg > 0, lax.rsqrt(safe), 0.0)


def _tc_mm_body(x_ref, w_ref, y_ref):
    y_ref[...] = jnp.dot(x_ref[...], w_ref[...],
                         preferred_element_type=jnp.float32)


def _tc1_body(xw_ref, degp_ref, y_ref):
    y_ref[...] = xw_ref[...] * _dis_from(degp_ref)


def _tc2_body(agg_ref, degp_ref, b_ref, w_ref, y_ref):
    dis = _dis_from(degp_ref)
    agg = agg_ref[:N] + agg_ref[NP:NP + N]
    h = jnp.maximum(agg * dis + b_ref[...][None, :], 0.0)
    y_ref[...] = jnp.dot(h, w_ref[...],
                         preferred_element_type=jnp.float32) * dis


def _tc3_body(agg_ref, degp_ref, b_ref, h_ref):
    dis = _dis_from(degp_ref)
    agg = agg_ref[:N] + agg_ref[NP:NP + N]
    h_ref[:N] = (agg * dis + b_ref[...][None, :]).astype(jnp.bfloat16)
    h_ref[N:] = jnp.zeros((NP - N, D), jnp.bfloat16)


_tc_mm = pl.pallas_call(_tc_mm_body,
                        out_shape=jax.ShapeDtypeStruct((N, D), jnp.float32))
_tc1 = pl.pallas_call(_tc1_body,
                      out_shape=jax.ShapeDtypeStruct((N, D), jnp.float32))
_tc2 = pl.pallas_call(_tc2_body,
                      out_shape=jax.ShapeDtypeStruct((N, D), jnp.float32))
_tc3 = pl.pallas_call(_tc3_body,
                      out_shape=jax.ShapeDtypeStruct((NP, D), jnp.bfloat16))


def kernel(x, edge_index, W1, b1, W2, b2):
    row = edge_index[0].astype(jnp.int32)
    col = edge_index[1].astype(jnp.int32)
    zeros16 = jnp.zeros((NPS, 16), jnp.float32)
    ones16 = jnp.ones((CHUNK, 16), jnp.float32)
    zerosD = jnp.zeros((CHUNK, D), jnp.float32)

    degp = _deg_kernel(col, zeros16, ones16)
    xw1 = _tc_mm(x, W1)
    y1 = _tc1(xw1, degp)
    agg1 = _agg_kernel(y1, row, col, zerosD)
    y2 = _tc2(agg1, degp, b1, W2)
    agg2 = _agg_kernel(y2, row, col, zerosD)
    h2 = _tc3(agg2, degp, b2)
    return _edge_kernel(h2, row, col)


# speedup vs baseline: 1.0038x; 1.0038x over previous
import functools

import jax
import jax.numpy as jnp
from jax import lax
from jax.experimental import pallas as pl
from jax.experimental.pallas import tpu as pltpu
from jax.experimental.pallas import tpu_sc as plsc

N = 10000
D = 128
E = 320000

NC = 2
NS = 16
NW = NC * NS
EPW = E // NW
CHUNK = 80
NCHUNK = EPW // CHUNK
NP = 10240
NPS = NP // NS
RING = 3
PEEL = NCHUNK % RING

_mesh = plsc.VectorSubcoreMesh(core_axis_name="c", subcore_axis_name="s")
_sc_params = pltpu.CompilerParams(use_tc_tiling_on_sc=False,
                                 needs_layout_passes=False)


def _dma_sems():
    return [pltpu.SemaphoreType.DMA for _ in range(RING)]


@functools.partial(
    pl.kernel,
    out_type=jax.ShapeDtypeStruct((NC * NP, 16), jnp.float32),
    mesh=_mesh,
    compiler_params=_sc_params,
    scratch_types=[
        pltpu.VMEM_SHARED((NP, 16), jnp.float32),
        [pltpu.VMEM((CHUNK,), jnp.int32) for _ in range(RING)],
        pltpu.VMEM((CHUNK, 16), jnp.float32),
        pltpu.VMEM((NPS, 16), jnp.float32),
        _dma_sems(),
    ],
)
def _deg_kernel(col_hbm, zeros_hbm, ones_hbm, out_hbm,
                degp, cidx, ones, zbuf, isem):
    cid = lax.axis_index("c")
    sid = lax.axis_index("s")
    wid = cid * NS + sid
    e0 = wid * EPW
    last = NCHUNK - 1
    pltpu.sync_copy(zeros_hbm, zbuf)
    pltpu.sync_copy(ones_hbm, ones)
    pltpu.sync_copy(zbuf, degp.at[pl.ds(sid * NPS, NPS)])
    plsc.subcore_barrier()

    def idx_start(i, s):
        pltpu.async_copy(col_hbm.at[pl.ds(e0 + i * CHUNK, CHUNK)],
                         cidx[s], isem[s])

    def idx_wait(s):
        pltpu.make_async_copy(col_hbm.at[pl.ds(e0, CHUNK)],
                              cidx[s], isem[s]).wait()

    for s in range(RING):
        idx_start(s, s)

    def step(i, s):
        idx_wait(s)
        pltpu.sync_copy(ones, degp.at[cidx[s]], add=True)
        idx_start(lax.min(i + RING, last), s)

    def body(io, carry):
        for b in range(RING):
            step(io * RING + b, b)
        return carry

    lax.fori_loop(0, (NCHUNK - PEEL) // RING, body, 0)
    for p in range(PEEL):
        i = NCHUNK - PEEL + p
        s = i % RING
        idx_wait(s)
        pltpu.sync_copy(ones, degp.at[cidx[s]], add=True)
    for p in range(PEEL, RING):
        idx_wait((NCHUNK - PEEL + p) % RING)
    plsc.subcore_barrier()
    pltpu.sync_copy(degp.at[pl.ds(sid * NPS, NPS)],
                    out_hbm.at[pl.ds(cid * NP + sid * NPS, NPS)])


@functools.partial(
    pl.kernel,
    out_type=jax.ShapeDtypeStruct((NC * NP, D), jnp.float32),
    mesh=_mesh,
    compiler_params=_sc_params,
    scratch_types=[
        pltpu.VMEM_SHARED((NP, D), jnp.float32),
        [pltpu.VMEM((CHUNK,), jnp.int32) for _ in range(RING)],
        [pltpu.VMEM((CHUNK,), jnp.int32) for _ in range(RING)],
        [pltpu.VMEM((CHUNK, D), jnp.float32) for _ in range(RING)],
        pltpu.VMEM((CHUNK, D), jnp.float32),
        _dma_sems(),
        _dma_sems(),
        _dma_sems(),
        _dma_sems(),
    ],
)
def _agg_kernel(y_hbm, row_hbm, col_hbm, zeros_hbm, out_hbm,
                acc, ridx, cidx, rows, zbuf, rsem, csem, gsem, ssem):
    cid = lax.axis_index("c")
    sid = lax.axis_index("s")
    wid = cid * NS + sid
    e0 = wid * EPW
    last = NCHUNK - 1
    pltpu.sync_copy(zeros_hbm, zbuf)
    for j in range(NPS // CHUNK):
        pltpu.sync_copy(zbuf, acc.at[pl.ds(sid * NPS + j * CHUNK, CHUNK)])
    plsc.subcore_barrier()

    def ridx_start(i, s):
        pltpu.async_copy(row_hbm.at[pl.ds(e0 + i * CHUNK, CHUNK)],
                         ridx[s], rsem[s])

    def ridx_wait(s):
        pltpu.make_async_copy(row_hbm.at[pl.ds(e0, CHUNK)],
                              ridx[s], rsem[s]).wait()

    def cidx_start(i, s):
        pltpu.async_copy(col_hbm.at[pl.ds(e0 + i * CHUNK, CHUNK)],
                         cidx[s], csem[s])

    def cidx_wait(s):
        pltpu.make_async_copy(col_hbm.at[pl.ds(e0, CHUNK)],
                              cidx[s], csem[s]).wait()

    def gather_start(s):
        pltpu.async_copy(y_hbm.at[ridx[s]], rows[s], gsem[s])

    def gather_wait(s):
        pltpu.make_async_copy(y_hbm.at[ridx[s]], rows[s], gsem[s]).wait()

    def scatter_start(s):
        pltpu.async_copy(rows[s], acc.at[cidx[s]], ssem[s], add=True)

    def scatter_wait(s):
        pltpu.make_async_copy(rows[s], acc.at[cidx[s]], ssem[s]).wait()

    for s in range(RING):
        ridx_start(s, s)
    cidx_start(0, 0)
    cidx_start(1, 1)
    ridx_wait(0)
    gather_start(0)
    ridx_wait(1)
    gather_start(1)

    gather_wait(0)
    ridx_wait(2)
    gather_start(2)
    cidx_wait(0)
    scatter_start(0)
    ridx_start(3, 0)
    cidx_start(2, 2)

    gather_wait(1)
    ridx_wait(0)
    scatter_wait(0)
    gather_start(0)
    cidx_wait(1)
    scatter_start(1)
    ridx_start(4, 1)
    cidx_start(3, 0)

    def step(i, s):
        s1 = (s + 2) % RING
        gather_wait(s)
        ridx_wait(s1)
        scatter_wait(s1)
        gather_start(s1)
        cidx_wait(s)
        scatter_start(s)
        ridx_start(lax.min(i + 3, last), s)
        cidx_start(lax.min(i + 2, last), s1)

    def body(io, carry):
        for b in range(RING):
            i = 2 + io * RING + b
            step(i, (2 + b) % RING)
        return carry

    lax.fori_loop(0, (NCHUNK - 2) // RING, body, 0)
    scatter_wait((NCHUNK - 1) % RING)
    gather_wait((NCHUNK + 1) % RING)
    gather_wait(NCHUNK % RING)
    ridx_wait((NCHUNK + 2) % RING)
    cidx_wait(NCHUNK % RING)
    cidx_wait((NCHUNK + 1) % RING)
    plsc.subcore_barrier()
    pltpu.sync_copy(acc.at[pl.ds(sid * NPS, NPS)],
                    out_hbm.at[pl.ds(cid * NP + sid * NPS, NPS)])


_EU = 4
RING_E = 3
PEEL_E = NCHUNK % RING_E


@functools.partial(
    pl.kernel,
    out_type=jax.ShapeDtypeStruct((E,), jnp.float32),
    mesh=_mesh,
    compiler_params=_sc_params,
    scratch_types=[
        pltpu.VMEM_SHARED((NP, D), jnp.bfloat16),
        [pltpu.VMEM((CHUNK,), jnp.int32) for _ in range(RING_E)],
        [pltpu.VMEM((CHUNK,), jnp.int32) for _ in range(RING_E)],
        [pltpu.VMEM((CHUNK, D), jnp.bfloat16) for _ in range(RING_E)],
        [pltpu.VMEM((CHUNK, D), jnp.bfloat16) for _ in range(RING_E)],
        [pltpu.VMEM((CHUNK,), jnp.float32) for _ in range(RING_E)],
        [pltpu.SemaphoreType.DMA for _ in range(RING_E)],
        [pltpu.SemaphoreType.DMA for _ in range(RING_E)],
        [pltpu.SemaphoreType.DMA for _ in range(RING_E)],
    ],
)
def _edge_kernel(h_hbm, row_hbm, col_hbm, out_hbm,
                 hsh, ridx, cidx, rr, rc, sbuf, isem, gsem, osem):
    cid = lax.axis_index("c")
    sid = lax.axis_index("s")
    wid = cid * NS + sid
    e0 = wid * EPW
    last = NCHUNK - 1
    lane = lax.iota(jnp.int32, 16)
    pltpu.sync_copy(h_hbm.at[pl.ds(sid * NPS, NPS)],
                    hsh.at[pl.ds(sid * NPS, NPS)])
    plsc.subcore_barrier()

    def idx_start(i, s):
        pltpu.async_copy(row_hbm.at[pl.ds(e0 + i * CHUNK, CHUNK)],
                         ridx[s], isem[s])
        pltpu.async_copy(col_hbm.at[pl.ds(e0 + i * CHUNK, CHUNK)],
                         cidx[s], isem[s])

    def idx_wait(s):
        pltpu.make_async_copy(row_hbm.at[pl.ds(e0, CHUNK)],
                              ridx[s], isem[s]).wait()
        pltpu.make_async_copy(col_hbm.at[pl.ds(e0, CHUNK)],
                              cidx[s], isem[s]).wait()

    def gather_start(s):
        pltpu.async_copy(hsh.at[ridx[s]], rr[s], gsem[s])
        pltpu.async_copy(hsh.at[cidx[s]], rc[s], gsem[s])

    def gather_wait(s):
        pltpu.make_async_copy(hsh.at[ridx[s]], rr[s], gsem[s]).wait()
        pltpu.make_async_copy(hsh.at[cidx[s]], rc[s], gsem[s]).wait()

    dnums = lax.GatherDimensionNumbers(
        offset_dims=(), collapsed_slice_dims=(0,), start_index_map=(0,))

    def allsum(v):
        for sh in (8, 4, 2, 1):
            p = lax.gather(v, (lane ^ sh)[:, None], dnums, slice_sizes=(1,),
                           mode=lax.GatherScatterMode.PROMISE_IN_BOUNDS)
            v = v + p
        return v

    himask = jnp.full((16,), -65536, jnp.int32)

    def split_bf16(x32):
        xi = plsc.bitcast(x32, jnp.int32)
        lo = plsc.bitcast(lax.shift_left(xi, 16), jnp.float32)
        hi = plsc.bitcast(xi & himask, jnp.float32)
        return lo, hi

    def compute(s):
        def group_body(g, carry2):
            def edge_body(t, acc16):
                for u in range(_EU):
                    e = g * 16 + t * _EU + u
                    v0 = jnp.zeros((16,), jnp.float32)
                    v1 = jnp.zeros((16,), jnp.float32)
                    for k in range(D // 32):
                        a0, a1 = split_bf16(rr[s][e, pl.ds(k * 32, 32)])
                        b0, b1 = split_bf16(rc[s][e, pl.ds(k * 32, 32)])
                        v0 = v0 + a0 * b0
                        v1 = v1 + a1 * b1
                    acc16 = jnp.where(lane == t * _EU + u,
                                      allsum(v0 + v1), acc16)
                return acc16

            z = lax.fori_loop(0, 16 // _EU, edge_body,
                              jnp.zeros((16,), jnp.float32))
            sbuf[s][pl.ds(g * 16, 16)] = 1.0 / (1.0 + jnp.exp(-z))
            return carry2

        lax.fori_loop(0, CHUNK // 16, group_body, 0)

    def out_start(i, s):
        pltpu.async_copy(sbuf[s], out_hbm.at[pl.ds(e0 + i * CHUNK, CHUNK)],
                         osem[s])

    def out_wait(s):
        pltpu.make_async_copy(sbuf[s], out_hbm.at[pl.ds(e0, CHUNK)],
                              osem[s]).wait()

    for s in range(RING_E):
        idx_start(s, s)
    for s in range(RING_E - 1):
        idx_wait(s)
        gather_start(s)

    def step(i, s, first):
        s1 = (s + RING_E - 1) % RING_E
        gather_wait(s)
        idx_wait(s1)
        gather_start(s1)
        if not first:
            out_wait(s)
        compute(s)
        out_start(i, s)
        idx_start(lax.min(i + RING_E, last), s)

    for s in range(RING_E):
        step(s, s, True)

    def body(io, carry):
        for b in range(RING_E):
            step(RING_E + io * RING_E + b, b, False)
        return carry

    lax.fori_loop(0, (NCHUNK - RING_E - PEEL_E) // RING_E, body, 0)
    for p in range(PEEL_E):
        i = NCHUNK - PEEL_E + p
        s = i % RING_E
        gather_wait(s)
        out_wait(s)
        compute(s)
        out_start(i, s)
    for p in range(PEEL_E, RING_E):
        idx_wait((NCHUNK - PEEL_E + p) % RING_E)
    for s in range(RING_E):
        out_wait((NCHUNK - PEEL_E - 1 + ((s + 1))) % RING_E)


def _dis_from(degp_ref):
    deg = degp_ref[:N, 0:1] + degp_ref[NP:NP + N, 0:1]
    safe = jnp.where(deg > 0, deg, 1.0)
    return jnp.where(deg > 0, lax.rsqrt(safe), 0.0)


def _tc_mm_body(x_ref, w_ref, y_ref):
    y_ref[...] = jnp.dot(x_ref[...], w_ref[...],
                         preferred_element_type=jnp.float32)


def _tc1_body(xw_ref, degp_ref, y_ref):
    y_ref[...] = xw_ref[...] * _dis_from(degp_ref)


def _tc2_body(agg_ref, degp_ref, b_ref, w_ref, y_ref):
    dis = _dis_from(degp_ref)
    agg = agg_ref[:N] + agg_ref[NP:NP + N]
    h = jnp.maximum(agg * dis + b_ref[...][None, :], 0.0)
    y_ref[...] = jnp.dot(h, w_ref[...],
                         preferred_element_type=jnp.float32) * dis


def _tc3_body(agg_ref, degp_ref, b_ref, h_ref):
    dis = _dis_from(degp_ref)
    agg = agg_ref[:N] + agg_ref[NP:NP + N]
    h_ref[:N] = (agg * dis + b_ref[...][None, :]).astype(jnp.bfloat16)
    h_ref[N:] = jnp.zeros((NP - N, D), jnp.bfloat16)


_tc_mm = pl.pallas_call(_tc_mm_body,
                        out_shape=jax.ShapeDtypeStruct((N, D), jnp.float32))
_tc1 = pl.pallas_call(_tc1_body,
                      out_shape=jax.ShapeDtypeStruct((N, D), jnp.float32))
_tc2 = pl.pallas_call(_tc2_body,
                      out_shape=jax.ShapeDtypeStruct((N, D), jnp.float32))
_tc3 = pl.pallas_call(_tc3_body,
                      out_shape=jax.ShapeDtypeStruct((NP, D), jnp.bfloat16))


def kernel(x, edge_index, W1, b1, W2, b2):
    row = edge_index[0].astype(jnp.int32)
    col = edge_index[1].astype(jnp.int32)
    zeros16 = jnp.zeros((NPS, 16), jnp.float32)
    ones16 = jnp.ones((CHUNK, 16), jnp.float32)
    zerosD = jnp.zeros((CHUNK, D), jnp.float32)

    degp = _deg_kernel(col, zeros16, ones16)
    xw1 = _tc_mm(x, W1)
    y1 = _tc1(xw1, degp)
    agg1 = _agg_kernel(y1, row, col, zerosD)
    y2 = _tc2(agg1, degp, b1, W2)
    agg2 = _agg_kernel(y2, row, col, zerosD)
    h2 = _tc3(agg2, degp, b2)
    return _edge_kernel(h2, row, col)


# speedup vs baseline: 1.0839x; 1.0798x over previous
import functools

import jax
import jax.numpy as jnp
from jax import lax
from jax.experimental import pallas as pl
from jax.experimental.pallas import tpu as pltpu
from jax.experimental.pallas import tpu_sc as plsc

N = 10000
D = 128
E = 320000

NC = 2
NS = 16
NW = NC * NS
EPW = E // NW
CHUNK = 80
NCHUNK = EPW // CHUNK
NP = 10240
NPS = NP // NS
RING = 3
PEEL = NCHUNK % RING

_mesh = plsc.VectorSubcoreMesh(core_axis_name="c", subcore_axis_name="s")
_sc_params = pltpu.CompilerParams(use_tc_tiling_on_sc=False,
                                 needs_layout_passes=False)


def _dma_sems():
    return [pltpu.SemaphoreType.DMA for _ in range(RING)]


@functools.partial(
    pl.kernel,
    out_type=jax.ShapeDtypeStruct((NC * NP, 16), jnp.float32),
    mesh=_mesh,
    compiler_params=_sc_params,
    scratch_types=[
        pltpu.VMEM_SHARED((NP, 16), jnp.float32),
        [pltpu.VMEM((CHUNK,), jnp.int32) for _ in range(RING)],
        pltpu.VMEM((CHUNK, 16), jnp.float32),
        pltpu.VMEM((NPS, 16), jnp.float32),
        _dma_sems(),
    ],
)
def _deg_kernel(col_hbm, zeros_hbm, ones_hbm, out_hbm,
                degp, cidx, ones, zbuf, isem):
    cid = lax.axis_index("c")
    sid = lax.axis_index("s")
    wid = cid * NS + sid
    e0 = wid * EPW
    last = NCHUNK - 1
    pltpu.sync_copy(zeros_hbm, zbuf)
    pltpu.sync_copy(ones_hbm, ones)
    pltpu.sync_copy(zbuf, degp.at[pl.ds(sid * NPS, NPS)])
    plsc.subcore_barrier()

    def idx_start(i, s):
        pltpu.async_copy(col_hbm.at[pl.ds(e0 + i * CHUNK, CHUNK)],
                         cidx[s], isem[s])

    def idx_wait(s):
        pltpu.make_async_copy(col_hbm.at[pl.ds(e0, CHUNK)],
                              cidx[s], isem[s]).wait()

    for s in range(RING):
        idx_start(s, s)

    def step(i, s):
        idx_wait(s)
        pltpu.sync_copy(ones, degp.at[cidx[s]], add=True)
        idx_start(lax.min(i + RING, last), s)

    def body(io, carry):
        for b in range(RING):
            step(io * RING + b, b)
        return carry

    lax.fori_loop(0, (NCHUNK - PEEL) // RING, body, 0)
    for p in range(PEEL):
        i = NCHUNK - PEEL + p
        s = i % RING
        idx_wait(s)
        pltpu.sync_copy(ones, degp.at[cidx[s]], add=True)
    for p in range(PEEL, RING):
        idx_wait((NCHUNK - PEEL + p) % RING)
    plsc.subcore_barrier()
    pltpu.sync_copy(degp.at[pl.ds(sid * NPS, NPS)],
                    out_hbm.at[pl.ds(cid * NP + sid * NPS, NPS)])


@functools.partial(
    pl.kernel,
    out_type=jax.ShapeDtypeStruct((NC * NP, D), jnp.float32),
    mesh=_mesh,
    compiler_params=_sc_params,
    scratch_types=[
        pltpu.VMEM_SHARED((NP, D), jnp.float32),
        [pltpu.VMEM((CHUNK,), jnp.int32) for _ in range(RING)],
        [pltpu.VMEM((CHUNK,), jnp.int32) for _ in range(RING)],
        [pltpu.VMEM((CHUNK, D), jnp.float32) for _ in range(RING)],
        pltpu.VMEM((CHUNK, D), jnp.float32),
        _dma_sems(),
        _dma_sems(),
        _dma_sems(),
        _dma_sems(),
    ],
)
def _agg_kernel(y_hbm, row_hbm, col_hbm, zeros_hbm, out_hbm,
                acc, ridx, cidx, rows, zbuf, rsem, csem, gsem, ssem):
    cid = lax.axis_index("c")
    sid = lax.axis_index("s")
    wid = cid * NS + sid
    e0 = wid * EPW
    last = NCHUNK - 1
    pltpu.sync_copy(zeros_hbm, zbuf)
    for j in range(NPS // CHUNK):
        pltpu.sync_copy(zbuf, acc.at[pl.ds(sid * NPS + j * CHUNK, CHUNK)])
    plsc.subcore_barrier()

    def ridx_start(i, s):
        pltpu.async_copy(row_hbm.at[pl.ds(e0 + i * CHUNK, CHUNK)],
                         ridx[s], rsem[s])

    def ridx_wait(s):
        pltpu.make_async_copy(row_hbm.at[pl.ds(e0, CHUNK)],
                              ridx[s], rsem[s]).wait()

    def cidx_start(i, s):
        pltpu.async_copy(col_hbm.at[pl.ds(e0 + i * CHUNK, CHUNK)],
                         cidx[s], csem[s])

    def cidx_wait(s):
        pltpu.make_async_copy(col_hbm.at[pl.ds(e0, CHUNK)],
                              cidx[s], csem[s]).wait()

    def gather_start(s):
        pltpu.async_copy(y_hbm.at[ridx[s]], rows[s], gsem[s])

    def gather_wait(s):
        pltpu.make_async_copy(y_hbm.at[ridx[s]], rows[s], gsem[s]).wait()

    def scatter_start(s):
        pltpu.async_copy(rows[s], acc.at[cidx[s]], ssem[s], add=True)

    def scatter_wait(s):
        pltpu.make_async_copy(rows[s], acc.at[cidx[s]], ssem[s]).wait()

    for s in range(RING):
        ridx_start(s, s)
    cidx_start(0, 0)
    cidx_start(1, 1)
    ridx_wait(0)
    gather_start(0)
    ridx_wait(1)
    gather_start(1)

    gather_wait(0)
    ridx_wait(2)
    gather_start(2)
    cidx_wait(0)
    scatter_start(0)
    ridx_start(3, 0)
    cidx_start(2, 2)

    gather_wait(1)
    ridx_wait(0)
    scatter_wait(0)
    gather_start(0)
    cidx_wait(1)
    scatter_start(1)
    ridx_start(4, 1)
    cidx_start(3, 0)

    def step(i, s):
        s1 = (s + 2) % RING
        gather_wait(s)
        ridx_wait(s1)
        scatter_wait(s1)
        gather_start(s1)
        cidx_wait(s)
        scatter_start(s)
        ridx_start(lax.min(i + 3, last), s)
        cidx_start(lax.min(i + 2, last), s1)

    def body(io, carry):
        for b in range(RING):
            i = 2 + io * RING + b
            step(i, (2 + b) % RING)
        return carry

    lax.fori_loop(0, (NCHUNK - 2) // RING, body, 0)
    scatter_wait((NCHUNK - 1) % RING)
    gather_wait((NCHUNK + 1) % RING)
    gather_wait(NCHUNK % RING)
    ridx_wait((NCHUNK + 2) % RING)
    cidx_wait(NCHUNK % RING)
    cidx_wait((NCHUNK + 1) % RING)
    plsc.subcore_barrier()
    pltpu.sync_copy(acc.at[pl.ds(sid * NPS, NPS)],
                    out_hbm.at[pl.ds(cid * NP + sid * NPS, NPS)])


_EU = 4
RING_E = 3
PEEL_E = NCHUNK % RING_E


@functools.partial(
    pl.kernel,
    out_type=jax.ShapeDtypeStruct((E,), jnp.float32),
    mesh=_mesh,
    compiler_params=_sc_params,
    scratch_types=[
        pltpu.VMEM_SHARED((NP, D), jnp.bfloat16),
        [pltpu.VMEM((CHUNK,), jnp.int32) for _ in range(RING_E)],
        [pltpu.VMEM((CHUNK,), jnp.int32) for _ in range(RING_E)],
        [pltpu.VMEM((CHUNK, D), jnp.bfloat16) for _ in range(RING_E)],
        [pltpu.VMEM((CHUNK, D), jnp.bfloat16) for _ in range(RING_E)],
        [pltpu.VMEM((CHUNK,), jnp.float32) for _ in range(RING_E)],
        [pltpu.SemaphoreType.DMA for _ in range(RING_E)],
        [pltpu.SemaphoreType.DMA for _ in range(RING_E)],
        [pltpu.SemaphoreType.DMA for _ in range(RING_E)],
    ],
)
def _edge_kernel(h_hbm, row_hbm, col_hbm, out_hbm,
                 hsh, ridx, cidx, rr, rc, sbuf, isem, gsem, osem):
    cid = lax.axis_index("c")
    sid = lax.axis_index("s")
    wid = cid * NS + sid
    e0 = wid * EPW
    last = NCHUNK - 1
    lane = lax.iota(jnp.int32, 16)
    pltpu.sync_copy(h_hbm.at[pl.ds(sid * NPS, NPS)],
                    hsh.at[pl.ds(sid * NPS, NPS)])
    plsc.subcore_barrier()

    def idx_start(i, s):
        pltpu.async_copy(row_hbm.at[pl.ds(e0 + i * CHUNK, CHUNK)],
                         ridx[s], isem[s])
        pltpu.async_copy(col_hbm.at[pl.ds(e0 + i * CHUNK, CHUNK)],
                         cidx[s], isem[s])

    def idx_wait(s):
        pltpu.make_async_copy(row_hbm.at[pl.ds(e0, CHUNK)],
                              ridx[s], isem[s]).wait()
        pltpu.make_async_copy(col_hbm.at[pl.ds(e0, CHUNK)],
                              cidx[s], isem[s]).wait()

    def gather_start(s):
        pltpu.async_copy(hsh.at[ridx[s]], rr[s], gsem[s])
        pltpu.async_copy(hsh.at[cidx[s]], rc[s], gsem[s])

    def gather_wait(s):
        pltpu.make_async_copy(hsh.at[ridx[s]], rr[s], gsem[s]).wait()
        pltpu.make_async_copy(hsh.at[cidx[s]], rc[s], gsem[s]).wait()

    dnums = lax.GatherDimensionNumbers(
        offset_dims=(), collapsed_slice_dims=(0,), start_index_map=(0,))

    def allsum(v):
        for sh in (8, 4, 2, 1):
            p = lax.gather(v, (lane ^ sh)[:, None], dnums, slice_sizes=(1,),
                           mode=lax.GatherScatterMode.PROMISE_IN_BOUNDS)
            v = v + p
        return v

    himask = jnp.full((16,), -65536, jnp.int32)

    def split_bf16(x32):
        xi = plsc.bitcast(x32, jnp.int32)
        lo = plsc.bitcast(lax.shift_left(xi, 16), jnp.float32)
        hi = plsc.bitcast(xi & himask, jnp.float32)
        return lo, hi

    def compute(s):
        def group_body(g, carry2):
            def edge_body(t, acc16):
                for u in range(_EU):
                    e = g * 16 + t * _EU + u
                    w0 = jnp.zeros((32,), jnp.bfloat16)
                    w1 = jnp.zeros((32,), jnp.bfloat16)
                    for k in range(0, D // 32, 2):
                        w0 = w0 + (rr[s][e, pl.ds(k * 32, 32)]
                                   * rc[s][e, pl.ds(k * 32, 32)])
                        w1 = w1 + (rr[s][e, pl.ds((k + 1) * 32, 32)]
                                   * rc[s][e, pl.ds((k + 1) * 32, 32)])
                    v0, v1 = split_bf16(w0 + w1)
                    acc16 = jnp.where(lane == t * _EU + u,
                                      allsum(v0 + v1), acc16)
                return acc16

            z = lax.fori_loop(0, 16 // _EU, edge_body,
                              jnp.zeros((16,), jnp.float32))
            sbuf[s][pl.ds(g * 16, 16)] = 1.0 / (1.0 + jnp.exp(-z))
            return carry2

        lax.fori_loop(0, CHUNK // 16, group_body, 0)

    def out_start(i, s):
        pltpu.async_copy(sbuf[s], out_hbm.at[pl.ds(e0 + i * CHUNK, CHUNK)],
                         osem[s])

    def out_wait(s):
        pltpu.make_async_copy(sbuf[s], out_hbm.at[pl.ds(e0, CHUNK)],
                              osem[s]).wait()

    for s in range(RING_E):
        idx_start(s, s)
    for s in range(RING_E - 1):
        idx_wait(s)
        gather_start(s)

    def step(i, s, first):
        s1 = (s + RING_E - 1) % RING_E
        gather_wait(s)
        idx_wait(s1)
        gather_start(s1)
        if not first:
            out_wait(s)
        compute(s)
        out_start(i, s)
        idx_start(lax.min(i + RING_E, last), s)

    for s in range(RING_E):
        step(s, s, True)

    def body(io, carry):
        for b in range(RING_E):
            step(RING_E + io * RING_E + b, b, False)
        return carry

    lax.fori_loop(0, (NCHUNK - RING_E - PEEL_E) // RING_E, body, 0)
    for p in range(PEEL_E):
        i = NCHUNK - PEEL_E + p
        s = i % RING_E
        gather_wait(s)
        out_wait(s)
        compute(s)
        out_start(i, s)
    for p in range(PEEL_E, RING_E):
        idx_wait((NCHUNK - PEEL_E + p) % RING_E)
    for s in range(RING_E):
        out_wait((NCHUNK - PEEL_E - 1 + ((s + 1))) % RING_E)


def _dis_from(degp_ref):
    deg = degp_ref[:N, 0:1] + degp_ref[NP:NP + N, 0:1]
    safe = jnp.where(deg > 0, deg, 1.0)
    return jnp.where(deg > 0, lax.rsqrt(safe), 0.0)


def _tc_mm_body(x_ref, w_ref, y_ref):
    y_ref[...] = jnp.dot(x_ref[...], w_ref[...],
                         preferred_element_type=jnp.float32)


def _tc1_body(xw_ref, degp_ref, y_ref):
    y_ref[...] = xw_ref[...] * _dis_from(degp_ref)


def _tc2_body(agg_ref, degp_ref, b_ref, w_ref, y_ref):
    dis = _dis_from(degp_ref)
    agg = agg_ref[:N] + agg_ref[NP:NP + N]
    h = jnp.maximum(agg * dis + b_ref[...][None, :], 0.0)
    y_ref[...] = jnp.dot(h, w_ref[...],
                         preferred_element_type=jnp.float32) * dis


def _tc3_body(agg_ref, degp_ref, b_ref, h_ref):
    dis = _dis_from(degp_ref)
    agg = agg_ref[:N] + agg_ref[NP:NP + N]
    h_ref[:N] = (agg * dis + b_ref[...][None, :]).astype(jnp.bfloat16)
    h_ref[N:] = jnp.zeros((NP - N, D), jnp.bfloat16)


_tc_mm = pl.pallas_call(_tc_mm_body,
                        out_shape=jax.ShapeDtypeStruct((N, D), jnp.float32))
_tc1 = pl.pallas_call(_tc1_body,
                      out_shape=jax.ShapeDtypeStruct((N, D), jnp.float32))
_tc2 = pl.pallas_call(_tc2_body,
                      out_shape=jax.ShapeDtypeStruct((N, D), jnp.float32))
_tc3 = pl.pallas_call(_tc3_body,
                      out_shape=jax.ShapeDtypeStruct((NP, D), jnp.bfloat16))


def kernel(x, edge_index, W1, b1, W2, b2):
    row = edge_index[0].astype(jnp.int32)
    col = edge_index[1].astype(jnp.int32)
    zeros16 = jnp.zeros((NPS, 16), jnp.float32)
    ones16 = jnp.ones((CHUNK, 16), jnp.float32)
    zerosD = jnp.zeros((CHUNK, D), jnp.float32)

    degp = _deg_kernel(col, zeros16, ones16)
    xw1 = _tc_mm(x, W1)
    y1 = _tc1(xw1, degp)
    agg1 = _agg_kernel(y1, row, col, zerosD)
    y2 = _tc2(agg1, degp, b1, W2)
    agg2 = _agg_kernel(y2, row, col, zerosD)
    h2 = _tc3(agg2, degp, b2)
    return _edge_kernel(h2, row, col)
